# baseline (speedup 1.0000x reference)
import functools

import jax
import jax.numpy as jnp
from jax import lax
from jax.experimental import pallas as pl
from jax.experimental.pallas import tpu as pltpu
from jax.experimental.pallas import tpu_sc as plsc

_START = 65000
_ROWS = 1000000
_BATCH = 16384
_DIM = 64

_INFO = plsc.get_sparse_core_info()
_NC = _INFO.num_cores
_NS = _INFO.num_subcores
_NW = _NC * _NS
_BPW = _BATCH // _NW

_mesh = plsc.VectorSubcoreMesh(core_axis_name="c", subcore_axis_name="s")


@functools.partial(
    pl.kernel,
    mesh=_mesh,
    out_type=jax.ShapeDtypeStruct((_BATCH, _DIM), jnp.float32),
    scratch_types=[
        pltpu.VMEM((_BPW,), jnp.int32),
        pltpu.VMEM((_BPW, _DIM), jnp.float32),
        pltpu.SemaphoreType.DMA,
    ],
)
def _gather_kernel(table_hbm, idx_hbm, out_hbm, idx_v, rows_v, sem):
    wid = lax.axis_index("s") * _NC + lax.axis_index("c")
    pltpu.sync_copy(idx_hbm.at[wid], idx_v)

    def body(g, _):
        vec = idx_v[pl.ds(g * 16, 16)]
        for j in range(16):
            pltpu.make_async_copy(
                table_hbm.at[pl.ds(vec[j], 1)],
                rows_v.at[pl.ds(g * 16 + j, 1)],
                sem,
            ).start()
        return 0

    lax.fori_loop(0, _BPW // 16, body, 0)
    pltpu.make_async_copy(
        table_hbm.at[pl.ds(0, _BPW)],
        rows_v,
        sem,
    ).wait()
    pltpu.sync_copy(rows_v, out_hbm.at[pl.ds(wid * _BPW, _BPW)])


def kernel(embedding_weight, batch):
    gene = lax.slice(embedding_weight, (_START, 0), (_START + _ROWS, _DIM))
    idx = batch.astype(jnp.int32).reshape(_NW, _BPW)
    return _gather_kernel(gene, idx)


# speedup vs baseline: 2.3803x; 2.3803x over previous
import functools

import jax
import jax.numpy as jnp
from jax import lax
from jax.experimental import pallas as pl
from jax.experimental.pallas import tpu as pltpu
from jax.experimental.pallas import tpu_sc as plsc

_START = 65000
_BATCH = 16384
_DIM = 64

_INFO = plsc.get_sparse_core_info()
_NC = _INFO.num_cores
_NS = _INFO.num_subcores
_NW = _NC * _NS
_BPW = _BATCH // _NW
_RING = 8

_mesh = plsc.VectorSubcoreMesh(core_axis_name="c", subcore_axis_name="s")


@functools.partial(
    pl.kernel,
    mesh=_mesh,
    out_type=jax.ShapeDtypeStruct((_NW, _BPW // 2, 2 * _DIM), jnp.float32),
    scratch_types=[
        pltpu.VMEM((_BPW,), jnp.int32),
        pltpu.VMEM((_RING, _DIM, 128), jnp.float32),
        pltpu.VMEM((_BPW // 2, 2 * _DIM), jnp.float32),
        *([pltpu.SemaphoreType.DMA] * _RING),
    ],
    compiler_params=pltpu.CompilerParams(needs_layout_passes=False),
)
def _gather_kernel(table_hbm, idx_hbm, out_hbm, idx_v, stage_v, rows_v, *sems):
    wid = lax.axis_index("s") * _NC + lax.axis_index("c")
    pltpu.sync_copy(idx_hbm.at[wid], idx_v)

    def splat_q(i):
        return plsc.load_gather(idx_v, [jnp.full((16,), i, jnp.int32)]) + _START

    def fire(o, j):
        qs = splat_q(o * _RING + j)
        col0 = pl.multiple_of((qs[0] >> 7) << 7, 128)
        pltpu.make_async_copy(
            table_hbm.at[:, pl.ds(col0, 128)], stage_v.at[j], sems[j]
        ).start()

    def process(o, j):
        i = o * _RING + j
        qs = splat_q(i)
        lane = lax.rem(qs, jnp.full((16,), 128, jnp.int32))
        pltpu.make_async_copy(
            table_hbm.at[:, pl.ds(0, 128)], stage_v.at[j], sems[j]
        ).wait()
        buf = stage_v.at[j]
        vrow = i >> 1
        off0 = (i & 1) * _DIM
        for g in range(4):
            rowi = lax.iota(jnp.int32, 16) + g * 16
            vals = plsc.load_gather(buf, [rowi, lane])
            rows_v[vrow, pl.ds(off0 + g * 16, 16)] = vals

    for j in range(_RING):
        fire(0, j)

    def body(o, _):
        for j in range(_RING):
            process(o - 1, j)
            fire(o, j)
        return 0

    lax.fori_loop(1, _BPW // _RING, body, 0)
    last = _BPW // _RING - 1
    for j in range(_RING):
        process(last, j)
    pltpu.sync_copy(rows_v, out_hbm.at[wid])


def kernel(embedding_weight, batch):
    idx = batch.astype(jnp.int32).reshape(_NW, _BPW)
    out3 = _gather_kernel(embedding_weight.T, idx)
    return out3.reshape(_BATCH, _DIM)


# speedup vs baseline: 2.5247x; 1.0607x over previous
import functools

import jax
import jax.numpy as jnp
from jax import lax
from jax.experimental import pallas as pl
from jax.experimental.pallas import tpu as pltpu
from jax.experimental.pallas import tpu_sc as plsc

_START = 65000
_BATCH = 16384
_DIM = 64

_INFO = plsc.get_sparse_core_info()
_NC = _INFO.num_cores
_NS = _INFO.num_subcores
_NW = _NC * _NS
_BPW = _BATCH // _NW
_RING = 10

_mesh = plsc.VectorSubcoreMesh(core_axis_name="c", subcore_axis_name="s")


@functools.partial(
    pl.kernel,
    mesh=_mesh,
    out_type=jax.ShapeDtypeStruct((_DIM, _BATCH), jnp.float32),
    scratch_types=[
        pltpu.VMEM((_BPW,), jnp.int32),
        pltpu.VMEM((_RING, _DIM, 128), jnp.float32),
        pltpu.VMEM((_DIM, _BPW), jnp.float32),
        *([pltpu.SemaphoreType.DMA] * _RING),
    ],
    compiler_params=pltpu.CompilerParams(needs_layout_passes=False),
)
def _gather_kernel(table_hbm, idx_hbm, out_hbm, idx_v, stage_v, cols_v, *sems):
    wid = lax.axis_index("s") * _NC + lax.axis_index("c")
    pltpu.sync_copy(idx_hbm.at[wid], idx_v)

    def splat_q(i):
        return plsc.load_gather(idx_v, [jnp.full((16,), i, jnp.int32)]) + _START

    def fire(o, j):
        qs = splat_q(o * _RING + j)
        col0 = pl.multiple_of((qs[0] >> 7) << 7, 128)
        pltpu.make_async_copy(
            table_hbm.at[:, pl.ds(col0, 128)], stage_v.at[j], sems[j]
        ).start()

    def process(o, j):
        i = o * _RING + j
        qs = splat_q(i)
        lane = lax.rem(qs, jnp.full((16,), 128, jnp.int32))
        pltpu.make_async_copy(
            table_hbm.at[:, pl.ds(0, 128)], stage_v.at[j], sems[j]
        ).wait()
        buf = stage_v.at[j]
        dst_col = jnp.full((16,), i, jnp.int32)
        for g in range(4):
            rowi = lax.iota(jnp.int32, 16) + g * 16
            vals = plsc.load_gather(buf, [rowi, lane])
            plsc.store_scatter(cols_v, [rowi, dst_col], vals)

    for j in range(_RING):
        fire(0, j)

    def body(o, _):
        for j in range(_RING):
            process(o - 1, j)
            fire(o, j)
        return 0

    n_rounds = _BPW // _RING
    lax.fori_loop(1, n_rounds, body, 0)
    for j in range(_RING):
        process(n_rounds - 1, j)
    for i in range(n_rounds * _RING, _BPW):
        j = i % _RING
        fire_o, fire_j = divmod(i, _RING)
        qs = splat_q(i)
        col0 = pl.multiple_of((qs[0] >> 7) << 7, 128)
        pltpu.make_async_copy(
            table_hbm.at[:, pl.ds(col0, 128)], stage_v.at[j], sems[j]
        ).start()
        lane = lax.rem(qs, jnp.full((16,), 128, jnp.int32))
        pltpu.make_async_copy(
            table_hbm.at[:, pl.ds(0, 128)], stage_v.at[j], sems[j]
        ).wait()
        buf = stage_v.at[j]
        dst_col = jnp.full((16,), i, jnp.int32)
        for g in range(4):
            rowi = lax.iota(jnp.int32, 16) + g * 16
            vals = plsc.load_gather(buf, [rowi, lane])
            plsc.store_scatter(cols_v, [rowi, dst_col], vals)
    pltpu.sync_copy(cols_v, out_hbm.at[:, pl.ds(wid * _BPW, _BPW)])


def kernel(embedding_weight, batch):
    idx = batch.astype(jnp.int32).reshape(_NW, _BPW)
    out_t = _gather_kernel(embedding_weight.T, idx)
    return out_t.T


# speedup vs baseline: 2.7804x; 1.1013x over previous
import functools

import jax
import jax.numpy as jnp
from jax import lax
from jax.experimental import pallas as pl
from jax.experimental.pallas import tpu as pltpu
from jax.experimental.pallas import tpu_sc as plsc

_START = 65000
_NGENE = 1000000
_BATCH = 16384
_DIM = 64

_T0 = _START // 128
_T1 = (_START + _NGENE - 1) // 128 + 1

_INFO = plsc.get_sparse_core_info()
_NC = _INFO.num_cores
_NS = _INFO.num_subcores
_NW = _NC * _NS
_TPW = -(-(_T1 - _T0) // _NW)
_RING = 3
_POOL = 256
_NVEC = _BATCH // 16

_mesh = plsc.VectorSubcoreMesh(core_axis_name="c", subcore_axis_name="s")


@functools.partial(
    pl.kernel,
    mesh=_mesh,
    out_type=jax.ShapeDtypeStruct((_BATCH, _DIM), jnp.float32),
    scratch_types=[
        pltpu.VMEM((_BATCH,), jnp.int32),
        pltpu.VMEM((_BATCH,), jnp.int32),
        pltpu.VMEM((_BATCH,), jnp.int32),
        pltpu.VMEM((256,), jnp.int32),
        pltpu.VMEM((256,), jnp.int32),
        pltpu.VMEM((256,), jnp.int32),
        pltpu.VMEM((_RING, _DIM, 128), jnp.float32),
        pltpu.VMEM((_POOL, _DIM), jnp.float32),
        *([pltpu.SemaphoreType.DMA] * _RING),
        pltpu.SemaphoreType.DMA,
    ],
    compiler_params=pltpu.CompilerParams(needs_layout_passes=False),
)
def _gather_kernel(
    table_hbm, idx_hbm, out_hbm,
    idx_v, sq_v, spos_v, cnt_v, off_v, cur_v, stage_v, pool_v, *sems,
):
    st_sems = sems[:_RING]
    row_sem = sems[_RING]
    wid = lax.axis_index("s") * _NC + lax.axis_index("c")
    lo = _T0 + wid * _TPW
    n_t = jnp.minimum(jnp.int32(_TPW), jnp.int32(_T1) - lo)

    pltpu.sync_copy(idx_hbm, idx_v)

    ones16 = jnp.ones((16,), jnp.int32)

    for c in range(16):
        cnt_v[pl.ds(c * 16, 16)] = jnp.zeros((16,), jnp.int32)

    def p1(v, carry):
        qv = idx_v[pl.ds(v * 16, 16)] + _START
        t = qv >> 7
        m = (t >= lo) & (t - lo < n_t)
        plsc.addupdate_scatter(cnt_v, [t - lo], ones16, mask=m)
        return carry

    lax.fori_loop(0, _NVEC, p1, 0)

    carry = jnp.int32(0)
    for c in range(16):
        b = cnt_v[pl.ds(c * 16, 16)]
        s = plsc.cumsum(b)
        start = s - b + carry
        off_v[pl.ds(c * 16, 16)] = start
        cur_v[pl.ds(c * 16, 16)] = start
        carry = carry + s[15]
    nh = carry

    def p3(v, carry):
        qv = idx_v[pl.ds(v * 16, 16)] + _START
        t = qv >> 7
        m = (t >= lo) & (t - lo < n_t)
        tl = t - lo
        dup, _ = plsc.scan_count(tl, m)
        base = plsc.load_gather(cur_v, [tl], mask=m)
        slot = base + dup - 1
        pos = jnp.full((16,), v * 16, jnp.int32) + lax.iota(jnp.int32, 16)
        plsc.store_scatter(sq_v, [slot], qv, mask=m)
        plsc.store_scatter(spos_v, [slot], pos, mask=m)
        plsc.addupdate_scatter(cur_v, [tl], ones16, mask=m)
        return carry

    lax.fori_loop(0, _NVEC, p3, 0)

    def flush(pbase, n):
        def emit(k, carry):
            e = pbase + k
            pos = plsc.load_gather(spos_v, [jnp.full((16,), e, jnp.int32)])[0]
            pltpu.make_async_copy(
                pool_v.at[pl.ds(k, 1)], out_hbm.at[pl.ds(pos, 1)], row_sem
            ).start()
            return carry

        lax.fori_loop(0, n, emit, 0)

        def drain(k, carry):
            pltpu.make_async_copy(
                out_hbm.at[pl.ds(0, 1)], pool_v.at[pl.ds(0, 1)], row_sem
            ).wait()
            return carry

        lax.fori_loop(0, n, drain, 0)

    def scalar_at(ref, i):
        return plsc.load_gather(ref, [jnp.full((16,), i, jnp.int32)])[0]

    def fire(tl, j):
        cnt = scalar_at(cnt_v, tl)

        @pl.when((tl < n_t) & (cnt > 0))
        def _():
            col0 = pl.multiple_of((lo + tl) * 128, 128)
            pltpu.make_async_copy(
                table_hbm.at[:, pl.ds(col0, 128)], stage_v.at[j], st_sems[j]
            ).start()

    def process(tl, j, pbase):
        cnt = scalar_at(cnt_v, tl)

        def with_tile():
            pltpu.make_async_copy(
                table_hbm.at[:, pl.ds(0, 128)], stage_v.at[j], st_sems[j]
            ).wait()
            off = scalar_at(off_v, tl)
            buf = stage_v.at[j]

            def entry(k, pb):
                e = off + k

                def do_flush():
                    flush(pb, jnp.int32(_POOL))
                    return pb + _POOL

                pb = lax.cond(e - pb >= _POOL, do_flush, lambda: pb)
                qs = plsc.load_gather(sq_v, [jnp.full((16,), e, jnp.int32)])
                lane = lax.rem(qs, jnp.full((16,), 128, jnp.int32))
                prow = e - pb
                for g in range(4):
                    rowi = lax.iota(jnp.int32, 16) + g * 16
                    vals = plsc.load_gather(buf, [rowi, lane])
                    pool_v[prow, pl.ds(g * 16, 16)] = vals
                return pb

            return lax.fori_loop(0, cnt, entry, pbase)

        return lax.cond((tl < n_t) & (cnt > 0), with_tile, lambda: pbase)

    for j in range(_RING):
        fire(jnp.int32(j), j)

    def round_body(g, pbase):
        for j in range(_RING):
            pbase = process((g - 1) * _RING + j, j, pbase)
            fire(g * _RING + j, j)
        return pbase

    n_rounds = -(-_TPW // _RING) + 1
    pbase = lax.fori_loop(1, n_rounds, round_body, jnp.int32(0))
    flush(pbase, nh - pbase)


def kernel(embedding_weight, batch):
    return _gather_kernel(embedding_weight.T, batch.astype(jnp.int32))


# speedup vs baseline: 2.8130x; 1.0117x over previous
import functools

import jax
import jax.numpy as jnp
from jax import lax
from jax.experimental import pallas as pl
from jax.experimental.pallas import tpu as pltpu
from jax.experimental.pallas import tpu_sc as plsc

_START = 65000
_NGENE = 1000000
_BATCH = 16384
_DIM = 64

_T0 = _START // 128
_T1 = (_START + _NGENE - 1) // 128 + 1

_INFO = plsc.get_sparse_core_info()
_NC = _INFO.num_cores
_NS = _INFO.num_subcores
_NW = _NC * _NS
_TPW = -(-(_T1 - _T0) // _NW)
_RING = 3
_POOL = 256
_NVEC = _BATCH // 16

_mesh = plsc.VectorSubcoreMesh(core_axis_name="c", subcore_axis_name="s")


@functools.partial(
    pl.kernel,
    mesh=_mesh,
    out_type=jax.ShapeDtypeStruct((_BATCH, _DIM), jnp.float32),
    scratch_types=[
        pltpu.VMEM((_BATCH,), jnp.int32),
        pltpu.VMEM((_BATCH,), jnp.int32),
        pltpu.VMEM((_BATCH,), jnp.int32),
        pltpu.VMEM((256,), jnp.int32),
        pltpu.VMEM((256,), jnp.int32),
        pltpu.VMEM((256,), jnp.int32),
        pltpu.VMEM((_RING, _DIM, 128), jnp.float32),
        pltpu.VMEM((_POOL, _DIM), jnp.float32),
        *([pltpu.SemaphoreType.DMA] * _RING),
        pltpu.SemaphoreType.DMA,
    ],
    compiler_params=pltpu.CompilerParams(needs_layout_passes=False),
)
def _gather_kernel(
    table_hbm, idx_hbm, out_hbm,
    idx_v, sq_v, spos_v, cnt_v, off_v, cur_v, stage_v, pool_v, *sems,
):
    st_sems = sems[:_RING]
    row_sem = sems[_RING]
    wid = lax.axis_index("s") * _NC + lax.axis_index("c")
    lo = _T0 + wid * _TPW
    n_t = jnp.minimum(jnp.int32(_TPW), jnp.int32(_T1) - lo)

    pltpu.sync_copy(idx_hbm, idx_v)

    ones16 = jnp.ones((16,), jnp.int32)

    for c in range(16):
        cnt_v[pl.ds(c * 16, 16)] = jnp.zeros((16,), jnp.int32)

    def p1(u, carry):
        for uu in range(4):
            v = u * 4 + uu
            qv = idx_v[pl.ds(v * 16, 16)] + _START
            t = qv >> 7
            m = (t >= lo) & (t - lo < n_t)
            plsc.addupdate_scatter(cnt_v, [t - lo], ones16, mask=m)
        return carry

    lax.fori_loop(0, _NVEC // 4, p1, 0)

    carry = jnp.int32(0)
    for c in range(16):
        b = cnt_v[pl.ds(c * 16, 16)]
        s = plsc.cumsum(b)
        start = s - b + carry
        off_v[pl.ds(c * 16, 16)] = start
        cur_v[pl.ds(c * 16, 16)] = start
        carry = carry + s[15]
    nh = carry

    def scalar_at(ref, i):
        return plsc.load_gather(ref, [jnp.full((16,), i, jnp.int32)])[0]

    def fire(tl, j):
        cnt = scalar_at(cnt_v, tl)

        @pl.when((tl < n_t) & (cnt > 0))
        def _():
            col0 = pl.multiple_of((lo + tl) * 128, 128)
            pltpu.make_async_copy(
                table_hbm.at[:, pl.ds(col0, 128)], stage_v.at[j], st_sems[j]
            ).start()

    for j in range(_RING):
        fire(jnp.int32(j), j)

    def p3(u, carry):
        for uu in range(4):
            v = u * 4 + uu
            qv = idx_v[pl.ds(v * 16, 16)] + _START
            t = qv >> 7
            m = (t >= lo) & (t - lo < n_t)
            tl = t - lo
            dup, _ = plsc.scan_count(tl, m)
            base = plsc.load_gather(cur_v, [tl], mask=m)
            slot = base + dup - 1
            pos = jnp.full((16,), v * 16, jnp.int32) + lax.iota(jnp.int32, 16)
            plsc.store_scatter(sq_v, [slot], qv, mask=m)
            plsc.store_scatter(spos_v, [slot], pos, mask=m)
            plsc.addupdate_scatter(cur_v, [tl], ones16, mask=m)
        return carry

    lax.fori_loop(0, _NVEC // 4, p3, 0)

    def flush(pbase, n):
        def emit(k, carry):
            e = pbase + k
            pos = plsc.load_gather(spos_v, [jnp.full((16,), e, jnp.int32)])[0]
            pltpu.make_async_copy(
                pool_v.at[pl.ds(k, 1)], out_hbm.at[pl.ds(pos, 1)], row_sem
            ).start()
            return carry

        lax.fori_loop(0, n, emit, 0)

        def drain(k, carry):
            pltpu.make_async_copy(
                out_hbm.at[pl.ds(0, 1)], pool_v.at[pl.ds(0, 1)], row_sem
            ).wait()
            return carry

        lax.fori_loop(0, n, drain, 0)

    def process(tl, j, pbase):
        cnt = scalar_at(cnt_v, tl)

        def with_tile():
            pltpu.make_async_copy(
                table_hbm.at[:, pl.ds(0, 128)], stage_v.at[j], st_sems[j]
            ).wait()
            off = scalar_at(off_v, tl)
            buf = stage_v.at[j]

            def entry(k, pb):
                e = off + k

                def do_flush():
                    flush(pb, jnp.int32(_POOL))
                    return pb + _POOL

                pb = lax.cond(e - pb >= _POOL, do_flush, lambda: pb)
                qs = plsc.load_gather(sq_v, [jnp.full((16,), e, jnp.int32)])
                lane = lax.rem(qs, jnp.full((16,), 128, jnp.int32))
                prow = e - pb
                for g in range(4):
                    rowi = lax.iota(jnp.int32, 16) + g * 16
                    vals = plsc.load_gather(buf, [rowi, lane])
                    pool_v[prow, pl.ds(g * 16, 16)] = vals
                return pb

            return lax.fori_loop(0, cnt, entry, pbase)

        return lax.cond((tl < n_t) & (cnt > 0), with_tile, lambda: pbase)

    def round_body(g, pbase):
        for j in range(_RING):
            pbase = process((g - 1) * _RING + j, j, pbase)
            fire(g * _RING + j, j)
        return pbase

    n_rounds = -(-_TPW // _RING) + 1
    pbase = lax.fori_loop(1, n_rounds, round_body, jnp.int32(0))
    flush(pbase, nh - pbase)


def kernel(embedding_weight, batch):
    return _gather_kernel(embedding_weight.T, batch.astype(jnp.int32))


# speedup vs baseline: 3.4667x; 1.2324x over previous
import functools

import jax
import jax.numpy as jnp
from jax import lax
from jax.experimental import pallas as pl
from jax.experimental.pallas import tpu as pltpu
from jax.experimental.pallas import tpu_sc as plsc

_START = 65000
_NGENE = 1000000
_BATCH = 16384
_DIM = 64

_T0 = _START // 128
_T1 = (_START + _NGENE - 1) // 128 + 1

_INFO = plsc.get_sparse_core_info()
_NC = _INFO.num_cores
_NS = _INFO.num_subcores
_NW = _NC * _NS
_TPW = -(-(_T1 - _T0) // _NW)
_RING = 6
_POOL = 192
_NVEC = _BATCH // 16

_mesh = plsc.VectorSubcoreMesh(core_axis_name="c", subcore_axis_name="s")


@functools.partial(
    pl.kernel,
    mesh=_mesh,
    out_type=jax.ShapeDtypeStruct((_BATCH, _DIM), jnp.float32),
    scratch_types=[
        pltpu.VMEM((_BATCH,), jnp.int32),
        pltpu.VMEM((_BATCH,), jnp.int32),
        pltpu.VMEM((_BATCH,), jnp.int32),
        pltpu.VMEM((256,), jnp.int32),
        pltpu.VMEM((256,), jnp.int32),
        pltpu.VMEM((256,), jnp.int32),
        pltpu.VMEM((_RING, _DIM, 128), jnp.float32),
        pltpu.VMEM((_POOL, _DIM), jnp.float32),
        *([pltpu.SemaphoreType.DMA] * _RING),
        pltpu.SemaphoreType.DMA,
    ],
    compiler_params=pltpu.CompilerParams(needs_layout_passes=False),
)
def _gather_kernel(
    table_hbm, idx_hbm, out_hbm,
    idx_v, sq_v, spos_v, cnt_v, off_v, cur_v, stage_v, pool_v, *sems,
):
    st_sems = sems[:_RING]
    row_sem = sems[_RING]
    wid = lax.axis_index("s") * _NC + lax.axis_index("c")
    lo = _T0 + wid * _TPW
    n_t = jnp.minimum(jnp.int32(_TPW), jnp.int32(_T1) - lo)

    pltpu.sync_copy(idx_hbm, idx_v)

    ones16 = jnp.ones((16,), jnp.int32)

    for c in range(16):
        cnt_v[pl.ds(c * 16, 16)] = jnp.zeros((16,), jnp.int32)

    def p1(u, carry):
        for uu in range(4):
            v = u * 4 + uu
            qv = idx_v[pl.ds(v * 16, 16)] + _START
            t = qv >> 7
            m = (t >= lo) & (t - lo < n_t)
            plsc.addupdate_scatter(cnt_v, [t - lo], ones16, mask=m)
        return carry

    lax.fori_loop(0, _NVEC // 4, p1, 0)

    carry = jnp.int32(0)
    for c in range(16):
        b = cnt_v[pl.ds(c * 16, 16)]
        s = plsc.cumsum(b)
        start = s - b + carry
        off_v[pl.ds(c * 16, 16)] = start
        cur_v[pl.ds(c * 16, 16)] = start
        carry = carry + s[15]
    nh = carry

    def scalar_at(ref, i):
        return plsc.load_gather(ref, [jnp.full((16,), i, jnp.int32)])[0]

    def fire(tl, j):
        cnt = scalar_at(cnt_v, tl)

        @pl.when((tl < n_t) & (cnt > 0))
        def _():
            col0 = pl.multiple_of((lo + tl) * 128, 128)
            pltpu.make_async_copy(
                table_hbm.at[:, pl.ds(col0, 128)], stage_v.at[j], st_sems[j]
            ).start()

    for j in range(_RING):
        fire(jnp.int32(j), j)

    def p3(u, carry):
        for uu in range(4):
            v = u * 4 + uu
            qv = idx_v[pl.ds(v * 16, 16)] + _START
            t = qv >> 7
            m = (t >= lo) & (t - lo < n_t)
            tl = t - lo
            dup, _ = plsc.scan_count(tl, m)
            base = plsc.load_gather(cur_v, [tl], mask=m)
            slot = base + dup - 1
            pos = jnp.full((16,), v * 16, jnp.int32) + lax.iota(jnp.int32, 16)
            plsc.store_scatter(sq_v, [slot], qv, mask=m)
            plsc.store_scatter(spos_v, [slot], pos, mask=m)
            plsc.addupdate_scatter(cur_v, [tl], ones16, mask=m)
        return carry

    lax.fori_loop(0, _NVEC // 4, p3, 0)

    def flush(pbase, n):
        def emit(k, carry):
            e = pbase + k
            pos = plsc.load_gather(spos_v, [jnp.full((16,), e, jnp.int32)])[0]
            pltpu.make_async_copy(
                pool_v.at[pl.ds(k, 1)], out_hbm.at[pl.ds(pos, 1)], row_sem
            ).start()
            return carry

        lax.fori_loop(0, n, emit, 0)

        def drain(k, carry):
            pltpu.make_async_copy(
                out_hbm.at[pl.ds(0, 1)], pool_v.at[pl.ds(0, 1)], row_sem
            ).wait()
            return carry

        lax.fori_loop(0, n, drain, 0)

    def process(tl, j, pbase):
        cnt = scalar_at(cnt_v, tl)

        def with_tile():
            pltpu.make_async_copy(
                table_hbm.at[:, pl.ds(0, 128)], stage_v.at[j], st_sems[j]
            ).wait()
            off = scalar_at(off_v, tl)
            buf = stage_v.at[j]

            def entry(k, pb):
                e = off + k

                def do_flush():
                    flush(pb, jnp.int32(_POOL))
                    return pb + _POOL

                pb = lax.cond(e - pb >= _POOL, do_flush, lambda: pb)
                qs = plsc.load_gather(sq_v, [jnp.full((16,), e, jnp.int32)])
                lane = lax.rem(qs, jnp.full((16,), 128, jnp.int32))
                prow = e - pb
                for g in range(4):
                    rowi = lax.iota(jnp.int32, 16) + g * 16
                    vals = plsc.load_gather(buf, [rowi, lane])
                    pool_v[prow, pl.ds(g * 16, 16)] = vals
                return pb

            return lax.fori_loop(0, cnt, entry, pbase)

        return lax.cond((tl < n_t) & (cnt > 0), with_tile, lambda: pbase)

    def round_body(g, pbase):
        for j in range(_RING):
            pbase = process((g - 1) * _RING + j, j, pbase)
            fire(g * _RING + j, j)
        return pbase

    n_rounds = -(-_TPW // _RING) + 1
    pbase = lax.fori_loop(1, n_rounds, round_body, jnp.int32(0))
    flush(pbase, nh - pbase)


def kernel(embedding_weight, batch):
    return _gather_kernel(embedding_weight.T, batch.astype(jnp.int32))


# speedup vs baseline: 3.6100x; 1.0414x over previous
import functools

import jax
import jax.numpy as jnp
from jax import lax
from jax.experimental import pallas as pl
from jax.experimental.pallas import tpu as pltpu
from jax.experimental.pallas import tpu_sc as plsc

_START = 65000
_NGENE = 1000000
_BATCH = 16384
_DIM = 64

_T0 = _START // 128
_T1 = (_START + _NGENE - 1) // 128 + 1

_INFO = plsc.get_sparse_core_info()
_NC = _INFO.num_cores
_NS = _INFO.num_subcores
_NW = _NC * _NS
_TPW = -(-(_T1 - _T0) // _NW)
_RING = 8
_POOL = 128
_NVEC = _BATCH // 16

_mesh = plsc.VectorSubcoreMesh(core_axis_name="c", subcore_axis_name="s")


@functools.partial(
    pl.kernel,
    mesh=_mesh,
    out_type=jax.ShapeDtypeStruct((_BATCH, _DIM), jnp.float32),
    scratch_types=[
        pltpu.VMEM((_BATCH,), jnp.int32),
        pltpu.VMEM((_BATCH,), jnp.int32),
        pltpu.VMEM((256,), jnp.int32),
        pltpu.VMEM((256,), jnp.int32),
        pltpu.VMEM((256,), jnp.int32),
        pltpu.VMEM((_RING, _DIM, 128), jnp.float32),
        pltpu.VMEM((_POOL, _DIM), jnp.float32),
        *([pltpu.SemaphoreType.DMA] * _RING),
        pltpu.SemaphoreType.DMA,
    ],
    compiler_params=pltpu.CompilerParams(needs_layout_passes=False),
)
def _gather_kernel(
    table_hbm, idx_hbm, out_hbm,
    idx_v, sq_v, cnt_v, off_v, cur_v, stage_v, pool_v, *sems,
):
    st_sems = sems[:_RING]
    row_sem = sems[_RING]
    wid = lax.axis_index("s") * _NC + lax.axis_index("c")
    lo = _T0 + wid * _TPW
    n_t = jnp.minimum(jnp.int32(_TPW), jnp.int32(_T1) - lo)

    pltpu.sync_copy(idx_hbm, idx_v)

    ones16 = jnp.ones((16,), jnp.int32)

    for c in range(16):
        cnt_v[pl.ds(c * 16, 16)] = jnp.zeros((16,), jnp.int32)

    def p1(u, carry):
        for uu in range(4):
            v = u * 4 + uu
            qv = idx_v[pl.ds(v * 16, 16)] + _START
            t = qv >> 7
            m = (t >= lo) & (t - lo < n_t)
            plsc.addupdate_scatter(cnt_v, [t - lo], ones16, mask=m)
        return carry

    lax.fori_loop(0, _NVEC // 4, p1, 0)

    carry = jnp.int32(0)
    for c in range(16):
        b = cnt_v[pl.ds(c * 16, 16)]
        s = plsc.cumsum(b)
        start = s - b + carry
        off_v[pl.ds(c * 16, 16)] = start
        cur_v[pl.ds(c * 16, 16)] = start
        carry = carry + s[15]
    nh = carry

    def scalar_at(ref, i):
        return plsc.load_gather(ref, [jnp.full((16,), i, jnp.int32)])[0]

    def fire(tl, j):
        cnt = scalar_at(cnt_v, tl)

        @pl.when((tl < n_t) & (cnt > 0))
        def _():
            col0 = pl.multiple_of((lo + tl) * 128, 128)
            pltpu.make_async_copy(
                table_hbm.at[:, pl.ds(col0, 128)], stage_v.at[j], st_sems[j]
            ).start()

    for j in range(_RING):
        fire(jnp.int32(j), j)

    def p3(u, carry):
        for uu in range(4):
            v = u * 4 + uu
            qv = idx_v[pl.ds(v * 16, 16)] + _START
            t = qv >> 7
            m = (t >= lo) & (t - lo < n_t)
            tl = t - lo
            dup, _ = plsc.scan_count(tl, m)
            base = plsc.load_gather(cur_v, [tl], mask=m)
            slot = base + dup - 1
            pos = jnp.full((16,), v * 16, jnp.int32) + lax.iota(jnp.int32, 16)
            packed = ((qv & 127) << 14) | pos
            plsc.store_scatter(sq_v, [slot], packed, mask=m)
            plsc.addupdate_scatter(cur_v, [tl], ones16, mask=m)
        return carry

    lax.fori_loop(0, _NVEC // 4, p3, 0)

    def flush(pbase, n):
        def emit(k, carry):
            e = pbase + k
            pos = plsc.load_gather(sq_v, [jnp.full((16,), e, jnp.int32)])[0] & 16383
            pltpu.make_async_copy(
                pool_v.at[pl.ds(k, 1)], out_hbm.at[pl.ds(pos, 1)], row_sem
            ).start()
            return carry

        lax.fori_loop(0, n, emit, 0)

        def drain(k, carry):
            pltpu.make_async_copy(
                out_hbm.at[pl.ds(0, 1)], pool_v.at[pl.ds(0, 1)], row_sem
            ).wait()
            return carry

        lax.fori_loop(0, n, drain, 0)

    def process(tl, j, pbase):
        cnt = scalar_at(cnt_v, tl)

        def with_tile():
            pltpu.make_async_copy(
                table_hbm.at[:, pl.ds(0, 128)], stage_v.at[j], st_sems[j]
            ).wait()
            off = scalar_at(off_v, tl)
            buf = stage_v.at[j]

            def entry(k, pb):
                e = off + k

                def do_flush():
                    flush(pb, jnp.int32(_POOL))
                    return pb + _POOL

                pb = lax.cond(e - pb >= _POOL, do_flush, lambda: pb)
                pk = plsc.load_gather(sq_v, [jnp.full((16,), e, jnp.int32)])
                lane = (pk >> 14) & 127
                prow = e - pb
                for g in range(4):
                    rowi = lax.iota(jnp.int32, 16) + g * 16
                    vals = plsc.load_gather(buf, [rowi, lane])
                    pool_v[prow, pl.ds(g * 16, 16)] = vals
                return pb

            return lax.fori_loop(0, cnt, entry, pbase)

        return lax.cond((tl < n_t) & (cnt > 0), with_tile, lambda: pbase)

    def round_body(g, pbase):
        for j in range(_RING):
            pbase = process((g - 1) * _RING + j, j, pbase)
            fire(g * _RING + j, j)
        return pbase

    n_rounds = -(-_TPW // _RING) + 1
    pbase = lax.fori_loop(1, n_rounds, round_body, jnp.int32(0))
    flush(pbase, nh - pbase)


def kernel(embedding_weight, batch):
    return _gather_kernel(embedding_weight.T, batch.astype(jnp.int32))


# speedup vs baseline: 3.6676x; 1.0160x over previous
import functools

import jax
import jax.numpy as jnp
from jax import lax
from jax.experimental import pallas as pl
from jax.experimental.pallas import tpu as pltpu
from jax.experimental.pallas import tpu_sc as plsc

_START = 65000
_NGENE = 1000000
_BATCH = 16384
_DIM = 64

_T0 = _START // 128
_T1 = (_START + _NGENE - 1) // 128 + 1

_INFO = plsc.get_sparse_core_info()
_NC = _INFO.num_cores
_NS = _INFO.num_subcores
_NW = _NC * _NS
_TPW = -(-(_T1 - _T0) // _NW)
_RING = 9
_POOL = 128
_NVEC = _BATCH // 16

_mesh = plsc.VectorSubcoreMesh(core_axis_name="c", subcore_axis_name="s")


@functools.partial(
    pl.kernel,
    mesh=_mesh,
    out_type=jax.ShapeDtypeStruct((_BATCH, _DIM), jnp.float32),
    scratch_types=[
        pltpu.VMEM((_BATCH,), jnp.int32),
        pltpu.VMEM((_BATCH,), jnp.int32),
        pltpu.VMEM((256,), jnp.int32),
        pltpu.VMEM((256,), jnp.int32),
        pltpu.VMEM((256,), jnp.int32),
        pltpu.VMEM((_RING, _DIM, 128), jnp.float32),
        pltpu.VMEM((_POOL, _DIM), jnp.float32),
        *([pltpu.SemaphoreType.DMA] * _RING),
        pltpu.SemaphoreType.DMA,
    ],
    compiler_params=pltpu.CompilerParams(needs_layout_passes=False),
)
def _gather_kernel(
    table_hbm, idx_hbm, out_hbm,
    idx_v, sq_v, cnt_v, off_v, cur_v, stage_v, pool_v, *sems,
):
    st_sems = sems[:_RING]
    row_sem = sems[_RING]
    wid = lax.axis_index("s") * _NC + lax.axis_index("c")
    lo = _T0 + wid * _TPW
    n_t = jnp.minimum(jnp.int32(_TPW), jnp.int32(_T1) - lo)

    pltpu.sync_copy(idx_hbm, idx_v)

    ones16 = jnp.ones((16,), jnp.int32)

    for c in range(16):
        cnt_v[pl.ds(c * 16, 16)] = jnp.zeros((16,), jnp.int32)

    def p1(u, carry):
        for uu in range(4):
            v = u * 4 + uu
            qv = idx_v[pl.ds(v * 16, 16)] + _START
            t = qv >> 7
            m = (t >= lo) & (t - lo < n_t)
            plsc.addupdate_scatter(cnt_v, [t - lo], ones16, mask=m)
        return carry

    lax.fori_loop(0, _NVEC // 4, p1, 0)

    carry = jnp.int32(0)
    for c in range(16):
        b = cnt_v[pl.ds(c * 16, 16)]
        s = plsc.cumsum(b)
        start = s - b + carry
        off_v[pl.ds(c * 16, 16)] = start
        cur_v[pl.ds(c * 16, 16)] = start
        carry = carry + s[15]
    nh = carry

    def scalar_at(ref, i):
        return plsc.load_gather(ref, [jnp.full((16,), i, jnp.int32)])[0]

    def fire(tl, j):
        cnt = scalar_at(cnt_v, tl)

        @pl.when((tl < n_t) & (cnt > 0))
        def _():
            col0 = pl.multiple_of((lo + tl) * 128, 128)
            pltpu.make_async_copy(
                table_hbm.at[:, pl.ds(col0, 128)], stage_v.at[j], st_sems[j]
            ).start()

    for j in range(_RING):
        fire(jnp.int32(j), j)

    def p3(u, carry):
        for uu in range(4):
            v = u * 4 + uu
            qv = idx_v[pl.ds(v * 16, 16)] + _START
            t = qv >> 7
            m = (t >= lo) & (t - lo < n_t)
            tl = t - lo
            dup, _ = plsc.scan_count(tl, m)
            base = plsc.load_gather(cur_v, [tl], mask=m)
            slot = base + dup - 1
            pos = jnp.full((16,), v * 16, jnp.int32) + lax.iota(jnp.int32, 16)
            packed = ((qv & 127) << 14) | pos
            plsc.store_scatter(sq_v, [slot], packed, mask=m)
            plsc.addupdate_scatter(cur_v, [tl], ones16, mask=m)
        return carry

    lax.fori_loop(0, _NVEC // 4, p3, 0)

    def flush(pbase, n):
        def emit(k, carry):
            e = pbase + k
            pos = plsc.load_gather(sq_v, [jnp.full((16,), e, jnp.int32)])[0] & 16383
            pltpu.make_async_copy(
                pool_v.at[pl.ds(k, 1)], out_hbm.at[pl.ds(pos, 1)], row_sem
            ).start()
            return carry

        lax.fori_loop(0, n, emit, 0)

        def drain(k, carry):
            pltpu.make_async_copy(
                out_hbm.at[pl.ds(0, 1)], pool_v.at[pl.ds(0, 1)], row_sem
            ).wait()
            return carry

        lax.fori_loop(0, n, drain, 0)

    def process(tl, j, pbase):
        cnt = scalar_at(cnt_v, tl)

        def with_tile():
            pltpu.make_async_copy(
                table_hbm.at[:, pl.ds(0, 128)], stage_v.at[j], st_sems[j]
            ).wait()
            off = scalar_at(off_v, tl)
            buf = stage_v.at[j]

            def entry(k, pb):
                e = off + k

                def do_flush():
                    flush(pb, jnp.int32(_POOL))
                    return pb + _POOL

                pb = lax.cond(e - pb >= _POOL, do_flush, lambda: pb)
                pk = plsc.load_gather(sq_v, [jnp.full((16,), e, jnp.int32)])
                lane = (pk >> 14) & 127
                prow = e - pb
                for g in range(4):
                    rowi = lax.iota(jnp.int32, 16) + g * 16
                    vals = plsc.load_gather(buf, [rowi, lane])
                    pool_v[prow, pl.ds(g * 16, 16)] = vals
                return pb

            return lax.fori_loop(0, cnt, entry, pbase)

        return lax.cond((tl < n_t) & (cnt > 0), with_tile, lambda: pbase)

    def round_body(g, pbase):
        for j in range(_RING):
            pbase = process((g - 1) * _RING + j, j, pbase)
            fire(g * _RING + j, j)
        return pbase

    n_rounds = -(-_TPW // _RING) + 1
    pbase = lax.fori_loop(1, n_rounds, round_body, jnp.int32(0))
    flush(pbase, nh - pbase)


def kernel(embedding_weight, batch):
    return _gather_kernel(embedding_weight.T, batch.astype(jnp.int32))
